# Initial kernel scaffold; baseline (speedup 1.0000x reference)
import jax
import jax.numpy as jnp
from jax.experimental import pallas as pl


def kernel(x, proposals, gt_bbox, W_cls, W_bbox, gt_labels):
    raise NotImplementedError("write your pallas kernel here")


# speedup vs baseline: 1.1722x; 1.1722x over previous
import jax
import jax.numpy as jnp
from jax.experimental import pallas as pl
from jax.experimental.pallas import tpu as pltpu

N = 20000
G = 8
D = 2048
NUM_CLASSES = 30
C_MM = NUM_CLASSES + 1 + NUM_CLASSES * 4
C_OUT = C_MM + NUM_CLASSES + 1 + 4
BN = 1000


def _body(x_ref, p_ref, gt_ref, lab_ref, w_ref, out_ref):
    mm = jnp.dot(x_ref[...], w_ref[...], preferred_element_type=jnp.float32)

    p = p_ref[...]
    px1 = p[:, 0:1]
    py1 = p[:, 1:2]
    px2 = p[:, 2:3]
    py2 = p[:, 3:4]
    area_b = (px2 - px1 + 1.0) * (py2 - py1 + 1.0)

    col_ids = jax.lax.broadcasted_iota(jnp.int32, (BN, NUM_CLASSES), 1)
    cls_ov = jnp.zeros((BN, NUM_CLASSES), jnp.float32)
    best_iou = jnp.full((BN, 1), -1.0, jnp.float32)
    bgx1 = jnp.zeros((BN, 1), jnp.float32)
    bgy1 = jnp.zeros((BN, 1), jnp.float32)
    bgx2 = jnp.zeros((BN, 1), jnp.float32)
    bgy2 = jnp.zeros((BN, 1), jnp.float32)

    for g in range(G):
        gx1 = gt_ref[g, 0]
        gy1 = gt_ref[g, 1]
        gx2 = gt_ref[g, 2]
        gy2 = gt_ref[g, 3]
        area_g = (gx2 - gx1 + 1.0) * (gy2 - gy1 + 1.0)
        iw = jnp.maximum(jnp.minimum(px2, gx2) - jnp.maximum(px1, gx1) + 1.0, 0.0)
        ih = jnp.maximum(jnp.minimum(py2, gy2) - jnp.maximum(py1, gy1) + 1.0, 0.0)
        inter = iw * ih
        union = area_b + area_g - inter
        iou_g = inter / jnp.maximum(union, 1e-6)

        lbl = lab_ref[0, g]
        cls_ov = jnp.maximum(cls_ov, jnp.where(col_ids == lbl, iou_g, 0.0))

        upd = iou_g > best_iou
        best_iou = jnp.where(upd, iou_g, best_iou)
        bgx1 = jnp.where(upd, gx1, bgx1)
        bgy1 = jnp.where(upd, gy1, bgy1)
        bgx2 = jnp.where(upd, gx2, bgx2)
        bgy2 = jnp.where(upd, gy2, bgy2)

    src_w = jnp.maximum(px2 - px1, 1e-3)
    src_h = jnp.maximum(py2 - py1, 1e-3)
    src_cx = px1 + 0.5 * src_w
    src_cy = py1 + 0.5 * src_h
    gt_w = jnp.maximum(bgx2 - bgx1, 1e-3)
    gt_h = jnp.maximum(bgy2 - bgy1, 1e-3)
    gt_cx = bgx1 + 0.5 * gt_w
    gt_cy = bgy1 + 0.5 * gt_h
    tx = (gt_cx - src_cx) / src_w
    ty = (gt_cy - src_cy) / src_h
    tw = jnp.log(gt_w / src_w)
    th = jnp.log(gt_h / src_h)

    out_ref[...] = jnp.concatenate(
        [mm, cls_ov, best_iou, tx, ty, tw, th], axis=1)


def kernel(x, proposals, gt_bbox, W_cls, W_bbox, gt_labels):
    w = jnp.concatenate([W_cls, W_bbox], axis=1)
    lab = gt_labels.astype(jnp.int32).reshape(1, G)
    grid = (N // BN,)
    return pl.pallas_call(
        _body,
        grid=grid,
        in_specs=[
            pl.BlockSpec((BN, D), lambda i: (i, 0)),
            pl.BlockSpec((BN, 4), lambda i: (i, 0)),
            pl.BlockSpec((G, 4), lambda i: (0, 0)),
            pl.BlockSpec((1, G), lambda i: (0, 0)),
            pl.BlockSpec((D, C_MM), lambda i: (0, 0)),
        ],
        out_specs=pl.BlockSpec((BN, C_OUT), lambda i: (i, 0)),
        out_shape=jax.ShapeDtypeStruct((N, C_OUT), jnp.float32),
        compiler_params=pltpu.CompilerParams(
            dimension_semantics=("parallel",),
        ),
    )(x, proposals, gt_bbox, lab, w)
